# Initial kernel scaffold; baseline (speedup 1.0000x reference)
import jax
import jax.numpy as jnp
from jax.experimental import pallas as pl


def kernel(user, positive, negative, edge_index, edge_values, user_table, item_table, fc_W, fc_b, fcg_W, fcg_b):
    raise NotImplementedError("write your pallas kernel here")


# speedup vs baseline: 1.0002x; 1.0002x over previous
import jax
import jax.numpy as jnp
from jax.experimental import pallas as pl

NUM_USERS = 25000
NUM_ITEMS = 25000
N = NUM_USERS + NUM_ITEMS
E = 800000
D = 64
G = 3
L = 3
REG_LAMBDA = 1e-4
B = 4096


def _leaky(x):
    return jnp.where(x >= 0, x, 0.01 * x)


def _spmm(row, col, vals, x):
    return jax.ops.segment_sum(x[col] * vals[:, None], row, num_segments=N)


def _loss_body(u_ref, p_ref, n_ref, eu_ref, ep_ref, en_ref, bpr_ref, reg_ref):
    u = u_ref[...]
    p = p_ref[...]
    n = n_ref[...]
    pos = jnp.sum(u * p, axis=1)
    neg = jnp.sum(u * n, axis=1)
    x = neg - pos
    sp = jnp.maximum(x, 0.0) + jnp.log1p(jnp.exp(-jnp.abs(x)))
    bpr_ref[...] = jnp.mean(sp).reshape(1, 1)
    reg = 0.5 * (jnp.sum(eu_ref[...] ** 2) + jnp.sum(ep_ref[...] ** 2)
                 + jnp.sum(en_ref[...] ** 2)) / B
    reg_ref[...] = (REG_LAMBDA * reg).reshape(1, 1)


def kernel(user, positive, negative, edge_index, edge_values, user_table,
           item_table, fc_W, fc_b, fcg_W, fcg_b):
    row = edge_index[0]
    col = edge_index[1]
    all_emb = jnp.concatenate([user_table, item_table], axis=0)
    side = _spmm(row, col, edge_values, all_emb)
    temp = _leaky((all_emb + side) @ fc_W + fc_b)
    group_scores = temp @ fcg_W + fcg_b
    a_top = jnp.max(group_scores, axis=1, keepdims=True)
    one_hot = (group_scores == a_top).astype(jnp.float32)
    user_group = one_hot[:NUM_USERS]
    item_group = jnp.ones((NUM_ITEMS, G), dtype=jnp.float32)
    group_embedding = jnp.concatenate([user_group, item_group], axis=0).T
    layer_sums = [G * all_emb]
    cur = [all_emb for _ in range(G)]
    for _layer in range(1, L):
        new = []
        for g in range(G):
            gvec = group_embedding[g]
            vals_g = edge_values * gvec[col] * gvec[row]
            new.append(_spmm(row, col, vals_g, cur[g]))
        cur = new
        layer_sums.append(sum(new))
    final = jnp.mean(jnp.stack(layer_sums, axis=1), axis=1)
    users_emb, items_emb = final[:NUM_USERS], final[NUM_USERS:]

    u = users_emb[user]
    p = items_emb[positive]
    n = items_emb[negative]
    ego_u = user_table[user]
    ego_p = item_table[positive]
    ego_n = item_table[negative]

    bpr, reg = pl.pallas_call(
        _loss_body,
        out_shape=[jax.ShapeDtypeStruct((1, 1), jnp.float32),
                   jax.ShapeDtypeStruct((1, 1), jnp.float32)],
    )(u, p, n, ego_u, ego_p, ego_n)
    return (bpr[0, 0], reg[0, 0])


# speedup vs baseline: 11.5277x; 11.5259x over previous
import functools

import jax
import jax.numpy as jnp
from jax import lax
from jax.experimental import pallas as pl
from jax.experimental.pallas import tpu as pltpu
from jax.experimental.pallas import tpu_sc as plsc

NUM_USERS = 25000
NUM_ITEMS = 25000
N = NUM_USERS + NUM_ITEMS
E = 800000
D = 64
G = 3
REG_LAMBDA = 1e-4
B = 4096

NSUB = 16
NCORE = 2
WIN = 512
PER_SUB = 50176
NWIN = PER_SUB // WIN
E_PAD = NSUB * PER_SUB
N_ACC = 50176
ROWS_PER_SUB = N_ACC // NSUB
ZCHUNK = 448


def _sc_mesh():
    return plsc.VectorSubcoreMesh(core_axis_name="c", subcore_axis_name="s")


def _make_sc_pass(table_rows, table_w, table_dtype, acc_dtype, core_off,
                  idx_mult, use_pat):
    need_sbuf = table_dtype == jnp.bfloat16 and acc_dtype == jnp.float32
    acc_w = table_w

    scratch = []
    if use_pat:
        scratch.append(pltpu.VMEM((WIN // 128, 128), jnp.int32))
    scratch += [
        pltpu.VMEM((WIN,), jnp.int32),
        pltpu.VMEM((WIN // 128, 128), jnp.int32),
        pltpu.VMEM((WIN,), jnp.float32),
        pltpu.VMEM((WIN,), jnp.int32),
        pltpu.VMEM((WIN, table_w), table_dtype),
    ]
    if need_sbuf:
        scratch.append(pltpu.VMEM((WIN, acc_w), jnp.float32))
    scratch += [
        pltpu.VMEM_SHARED((N_ACC, acc_w), acc_dtype),
        pltpu.SemaphoreType.DMA,
    ]

    out_type = jax.ShapeDtypeStruct((NCORE, N_ACC, acc_w), acc_dtype)

    @functools.partial(pl.kernel, out_type=out_type, mesh=_sc_mesh(),
                       scratch_types=scratch,
                       compiler_params=pltpu.CompilerParams(
                           use_tc_tiling_on_sc=False,
                           needs_layout_passes=False))
    def body(*args):
        if use_pat:
            table_hbm, col_hbm, row3d_hbm, v_hbm, pat_hbm, out_hbm = args[:6]
            refs = args[6:]
            patbuf = refs[0]
            refs = refs[1:]
        else:
            table_hbm, col_hbm, row3d_hbm, v_hbm, out_hbm = args[:5]
            refs = args[5:]
        colbuf, rowbuf2d, vbuf, idxbuf, gbuf = refs[:5]
        refs = refs[5:]
        if need_sbuf:
            sbuf = refs[0]
            refs = refs[1:]
        else:
            sbuf = gbuf
        acc, sem = refs

        c = lax.axis_index("c")
        s = lax.axis_index("s")
        base = s * PER_SUB

        zw = 32 if acc_dtype == jnp.bfloat16 else 16

        @pl.loop(0, ZCHUNK)
        def _z(i):
            for k in range(acc_w // zw):
                sbuf[i, pl.ds(k * zw, zw)] = jnp.zeros((zw,), acc_dtype)

        for j in range(ROWS_PER_SUB // ZCHUNK):
            r0 = pl.multiple_of(s * ROWS_PER_SUB + j * ZCHUNK, 8)
            pltpu.sync_copy(sbuf.at[pl.ds(0, ZCHUNK)], acc.at[pl.ds(r0, ZCHUNK)])
        plsc.subcore_barrier()

        @pl.loop(0, NWIN)
        def _w(w):
            off = pl.multiple_of(base + w * WIN, WIN)
            widx = s * NWIN + w
            pltpu.sync_copy(col_hbm.at[pl.ds(off, WIN)], colbuf)
            pltpu.sync_copy(row3d_hbm.at[widx], rowbuf2d)
            pltpu.sync_copy(v_hbm.at[pl.ds(off, WIN)], vbuf)
            if use_pat:
                for j in range(WIN // 128):
                    pltpu.async_copy(pat_hbm.at[rowbuf2d.at[j]],
                                     patbuf.at[j], sem).wait()

            coff = c * core_off

            @pl.loop(0, WIN // 16)
            def _i(k):
                col16 = colbuf[pl.ds(k * 16, 16)]
                idx16 = col16 * idx_mult + coff
                idxbuf[pl.ds(k * 16, 16)] = idx16

            if use_pat:
                for j in range(WIN // 128):
                    @pl.loop(0, 8)
                    def _p(k, j=j):
                        pat16 = patbuf[j, pl.ds(k * 16, 16)]
                        sl = pl.ds(j * 128 + k * 16, 16)
                        idxbuf[sl] = idxbuf[sl] + pat16

            pltpu.async_copy(table_hbm.at[idxbuf], gbuf, sem).wait()

            @pl.loop(0, WIN // 16)
            def _r(k16):
                v16 = vbuf[pl.ds(k16 * 16, 16)]
                for lane in range(16):
                    i = k16 * 16 + lane
                    sv = v16[lane]
                    if table_dtype == jnp.float32:
                        for k in range(table_w // 16):
                            sl = pl.ds(k * 16, 16)
                            gbuf[i, sl] = gbuf[i, sl] * sv
                    else:
                        for k in range(table_w // 32):
                            sl = pl.ds(k * 32, 32)
                            a, b2 = plsc.unpack(
                                gbuf[i, sl], format=plsc.PackFormat.INTERLEAVED)
                            a = a * sv
                            b2 = b2 * sv
                            if need_sbuf:
                                sbuf[i, pl.ds(k * 32, 16)] = a
                                sbuf[i, pl.ds(k * 32 + 16, 16)] = b2
                            else:
                                gbuf[i, sl] = plsc.pack(
                                    a, b2, format=plsc.PackFormat.INTERLEAVED)

            for j in range(WIN // 128):
                pltpu.sync_copy(sbuf.at[pl.ds(j * 128, 128)],
                                acc.at[rowbuf2d.at[j]], add=True)

        plsc.subcore_barrier()
        for j in range(ROWS_PER_SUB // ZCHUNK):
            r0 = pl.multiple_of(s * ROWS_PER_SUB + j * ZCHUNK, 8)
            pltpu.sync_copy(acc.at[pl.ds(r0, ZCHUNK)],
                            out_hbm.at[c].at[pl.ds(r0, ZCHUNK)])

    return body


_PASS_A = _make_sc_pass(2 * N, 32, jnp.float32, jnp.float32,
                        core_off=N, idx_mult=1, use_pat=False)
_PASS_B = _make_sc_pass(2 * N, 64, jnp.bfloat16, jnp.bfloat16,
                        core_off=N, idx_mult=1, use_pat=False)
_PASS_C = _make_sc_pass(2 * 7 * N, 32, jnp.bfloat16, jnp.float32,
                        core_off=7 * N, idx_mult=7, use_pat=True)


def _dense_body(emb_ref, side_ref, w_ref, b_ref, wg_ref, bg_ref, out_ref):
    x = emb_ref[...] + side_ref[...]
    t = jnp.dot(x, w_ref[...], preferred_element_type=jnp.float32)
    t = t + b_ref[...][None, :]
    t = jnp.where(t >= 0, t, 0.01 * t)
    sc = jnp.dot(t, wg_ref[...], preferred_element_type=jnp.float32)
    out_ref[...] = sc + bg_ref[...][None, :]


def _loss_body(u_ref, p_ref, n_ref, eu_ref, ep_ref, en_ref, bpr_ref, reg_ref):
    u = u_ref[...]
    p = p_ref[...]
    n = n_ref[...]
    pos = jnp.sum(u * p, axis=1)
    neg = jnp.sum(u * n, axis=1)
    x = neg - pos
    sp = jnp.maximum(x, 0.0) + jnp.log1p(jnp.exp(-jnp.abs(x)))
    bpr_ref[...] = jnp.mean(sp).reshape(1, 1)
    reg = 0.5 * (jnp.sum(eu_ref[...] ** 2) + jnp.sum(ep_ref[...] ** 2)
                 + jnp.sum(en_ref[...] ** 2)) / B
    reg_ref[...] = (REG_LAMBDA * reg).reshape(1, 1)


def kernel(user, positive, negative, edge_index, edge_values, user_table,
           item_table, fc_W, fc_b, fcg_W, fcg_b):
    row = edge_index[0].astype(jnp.int32)
    col = edge_index[1].astype(jnp.int32)
    v = edge_values

    npad = E_PAD - E
    padi = jnp.arange(npad, dtype=jnp.int32) % N
    rowp = jnp.concatenate([row, padi])
    colp = jnp.concatenate([col, padi])
    vp = jnp.concatenate([v, jnp.zeros((npad,), jnp.float32)])
    row2d = rowp.reshape(E_PAD // WIN, WIN // 128, 128)

    emb = jnp.concatenate([user_table, item_table], axis=0)

    emb2 = jnp.stack([emb[:, :32], emb[:, 32:]]).reshape(2 * N, 32)
    side2 = _PASS_A(emb2, colp, row2d, vp)
    side = jnp.concatenate([side2[0, :N], side2[1, :N]], axis=1)

    wg8 = jnp.zeros((D, 8), jnp.float32).at[:, :G].set(fcg_W)
    bg8 = jnp.zeros((8,), jnp.float32).at[:G].set(fcg_b)
    scores8 = pl.pallas_call(
        _dense_body,
        grid=(25,),
        in_specs=[
            pl.BlockSpec((2000, D), lambda i: (i, 0)),
            pl.BlockSpec((2000, D), lambda i: (i, 0)),
            pl.BlockSpec((D, D), lambda i: (0, 0)),
            pl.BlockSpec((D,), lambda i: (0,)),
            pl.BlockSpec((D, 8), lambda i: (0, 0)),
            pl.BlockSpec((8,), lambda i: (0,)),
        ],
        out_specs=pl.BlockSpec((2000, 8), lambda i: (i, 0)),
        out_shape=jax.ShapeDtypeStruct((N, 8), jnp.float32),
    )(emb, side, fc_W, fc_b, wg8, bg8)

    scores = scores8[:, :G]
    top = jnp.max(scores, axis=1, keepdims=True)
    m = (scores == top).astype(jnp.float32)
    is_item = (jnp.arange(N) >= NUM_USERS)[:, None]
    m = jnp.where(is_item, 1.0, m)
    pat = (m[:, 0] + 2.0 * m[:, 1] + 4.0 * m[:, 2]).astype(jnp.int32) - 1

    mx = m[:, :, None] * emb[:, None, :]
    s_halves = []
    for q in range(2):
        tq = []
        for k in range(2):
            feats = mx[:, :, q * 32 + k * 16: q * 32 + (k + 1) * 16]
            rowt = jnp.concatenate(
                [feats[:, 0], feats[:, 1], feats[:, 2],
                 jnp.zeros((N, 16), jnp.float32)], axis=1)
            tq.append(rowt)
        table_q = jnp.stack(tq).reshape(2 * N, 64).astype(jnp.bfloat16)
        out_q = _PASS_B(table_q, colp, row2d, vp)
        s_halves.append(out_q[:, :N].astype(jnp.float32))

    s_parts = []
    for q in range(2):
        oq = s_halves[q][:, :, :48].reshape(2, N, 3, 16)
        s_parts.append(jnp.concatenate([oq[0], oq[1]], axis=2))
    s = jnp.concatenate(s_parts, axis=2)

    cur = m[:, :, None] * s
    layer1 = cur.sum(axis=1)

    c0, c1, c2 = cur[:, 0], cur[:, 1], cur[:, 2]
    curp = jnp.stack([c0, c1, c0 + c1, c2, c0 + c2, c1 + c2, c0 + c1 + c2],
                     axis=1)
    curp2 = jnp.stack([curp[:, :, :32], curp[:, :, 32:]])
    curp_tbl = curp2.reshape(2 * 7 * N, 32).astype(jnp.bfloat16)
    l2 = _PASS_C(curp_tbl, colp, row2d, vp, pat)
    layer2 = jnp.concatenate([l2[0, :N], l2[1, :N]], axis=1)

    final = (G * emb + layer1 + layer2) * (1.0 / 3.0)
    users_emb, items_emb = final[:NUM_USERS], final[NUM_USERS:]

    u = users_emb[user]
    p = items_emb[positive]
    n = items_emb[negative]
    ego_u = user_table[user]
    ego_p = item_table[positive]
    ego_n = item_table[negative]

    bpr, reg = pl.pallas_call(
        _loss_body,
        out_shape=[jax.ShapeDtypeStruct((1, 1), jnp.float32),
                   jax.ShapeDtypeStruct((1, 1), jnp.float32)],
    )(u, p, n, ego_u, ego_p, ego_n)
    return (bpr[0, 0], reg[0, 0])


# speedup vs baseline: 12.2654x; 1.0640x over previous
import functools

import jax
import jax.numpy as jnp
from jax import lax
from jax.experimental import pallas as pl
from jax.experimental.pallas import tpu as pltpu
from jax.experimental.pallas import tpu_sc as plsc

NUM_USERS = 25000
NUM_ITEMS = 25000
N = NUM_USERS + NUM_ITEMS
E = 800000
D = 64
G = 3
REG_LAMBDA = 1e-4
B = 4096

NSUB = 16
NCORE = 2
WIN = 512
PER_SUB = 50176
NWIN = PER_SUB // WIN
E_PAD = NSUB * PER_SUB
N_ACC = 50176
ROWS_PER_SUB = N_ACC // NSUB
ZCHUNK = 448


def _sc_mesh():
    return plsc.VectorSubcoreMesh(core_axis_name="c", subcore_axis_name="s")


def _make_sc_pass(table_rows, table_w, table_dtype, acc_dtype, core_off,
                  idx_mult, use_pat, win):
    need_sbuf = table_dtype == jnp.bfloat16 and acc_dtype == jnp.float32
    acc_w = table_w
    nwin = PER_SUB // win
    nsc = win // 128
    zchunk = 448 if win >= 448 else (224 if win >= 224 else 112)
    nz = ROWS_PER_SUB // zchunk

    def dbl(t):
        return [t, t]

    scratch = []
    if use_pat:
        scratch += dbl(pltpu.VMEM((nsc, 128), jnp.int32))
        scratch += dbl(pltpu.SemaphoreType.DMA)
    scratch += (
        dbl(pltpu.VMEM((win,), jnp.int32))
        + dbl(pltpu.VMEM((nsc, 128), jnp.int32))
        + dbl(pltpu.VMEM((win,), jnp.float32))
        + dbl(pltpu.VMEM((win, table_w), table_dtype))
    )
    if need_sbuf:
        scratch += dbl(pltpu.VMEM((win, acc_w), jnp.float32))
    scratch += [
        pltpu.VMEM_SHARED((N_ACC, acc_w), acc_dtype),
        pltpu.SemaphoreType.DMA,
        pltpu.SemaphoreType.DMA,
        pltpu.SemaphoreType.DMA,
        pltpu.SemaphoreType.DMA,
    ]

    out_type = jax.ShapeDtypeStruct((NCORE, N_ACC, acc_w), acc_dtype)

    @functools.partial(pl.kernel, out_type=out_type, mesh=_sc_mesh(),
                       scratch_types=scratch,
                       compiler_params=pltpu.CompilerParams(
                           use_tc_tiling_on_sc=False,
                           needs_layout_passes=False))
    def body(*args):
        if use_pat:
            table_hbm, col_hbm, row3d_hbm, v_hbm, pat_hbm, out_hbm = args[:6]
            refs = list(args[6:])
            patbuf = [refs.pop(0), refs.pop(0)]
            psem = [refs.pop(0), refs.pop(0)]
        else:
            table_hbm, col_hbm, row3d_hbm, v_hbm, out_hbm = args[:5]
            refs = list(args[5:])
        colbuf = [refs.pop(0), refs.pop(0)]
        rowbuf2d = [refs.pop(0), refs.pop(0)]
        vbuf = [refs.pop(0), refs.pop(0)]
        gbuf = [refs.pop(0), refs.pop(0)]
        idxbuf = colbuf
        if need_sbuf:
            sbuf = [refs.pop(0), refs.pop(0)]
        else:
            sbuf = gbuf
        acc = refs.pop(0)
        gsem = [refs.pop(0), refs.pop(0)]
        ssem = [refs.pop(0), refs.pop(0)]

        c = lax.axis_index("c")
        s = lax.axis_index("s")
        base = s * PER_SUB

        zw = 32 if acc_dtype == jnp.bfloat16 else 16

        @pl.loop(0, zchunk)
        def _z(i):
            for k in range(acc_w // zw):
                sbuf[0][i, pl.ds(k * zw, zw)] = jnp.zeros((zw,), acc_dtype)

        for j in range(nz):
            r0 = pl.multiple_of(s * ROWS_PER_SUB + j * zchunk, 8)
            pltpu.sync_copy(sbuf[0].at[pl.ds(0, zchunk)],
                            acc.at[pl.ds(r0, zchunk)])
        plsc.subcore_barrier()

        def load_inputs(b, w):
            off = pl.multiple_of(base + w * win, win)
            widx = s * nwin + w
            pltpu.sync_copy(col_hbm.at[pl.ds(off, win)], colbuf[b])
            pltpu.sync_copy(row3d_hbm.at[widx], rowbuf2d[b])
            pltpu.sync_copy(v_hbm.at[pl.ds(off, win)], vbuf[b])
            if use_pat:
                return [pltpu.async_copy(pat_hbm.at[rowbuf2d[b].at[j]],
                                         patbuf[b].at[j], psem[b])
                        for j in range(nsc)]
            return None

        def compute_idx(b, hpat):
            coff = c * core_off

            @pl.loop(0, win // 16)
            def _i(k):
                col16 = colbuf[b][pl.ds(k * 16, 16)]
                idxbuf[b][pl.ds(k * 16, 16)] = col16 * idx_mult + coff

            if use_pat:
                for h in hpat:
                    h.wait()
                for j in range(nsc):
                    @pl.loop(0, 8)
                    def _p(k, j=j):
                        pat16 = patbuf[b][j, pl.ds(k * 16, 16)]
                        sl = pl.ds(j * 128 + k * 16, 16)
                        idxbuf[b][sl] = idxbuf[b][sl] + pat16

        def fire_gather(b):
            return pltpu.async_copy(table_hbm.at[idxbuf[b]], gbuf[b], gsem[b])

        def scale(b):
            @pl.loop(0, win // 16)
            def _r(k16):
                v16 = vbuf[b][pl.ds(k16 * 16, 16)]
                for lane in range(16):
                    i = k16 * 16 + lane
                    sv = v16[lane]
                    if table_dtype == jnp.float32:
                        for k in range(table_w // 16):
                            sl = pl.ds(k * 16, 16)
                            gbuf[b][i, sl] = gbuf[b][i, sl] * sv
                    else:
                        for k in range(table_w // 32):
                            sl = pl.ds(k * 32, 32)
                            a, b2 = plsc.unpack(
                                gbuf[b][i, sl],
                                format=plsc.PackFormat.INTERLEAVED)
                            a = a * sv
                            b2 = b2 * sv
                            if need_sbuf:
                                sbuf[b][i, pl.ds(k * 32, 16)] = a
                                sbuf[b][i, pl.ds(k * 32 + 16, 16)] = b2
                            else:
                                gbuf[b][i, sl] = plsc.pack(
                                    a, b2, format=plsc.PackFormat.INTERLEAVED)

        def fire_scatters(b):
            hs = []
            for j in range(nsc):
                hs.append(pltpu.async_copy(
                    sbuf[b].at[pl.ds(j * 128, 128)],
                    acc.at[rowbuf2d[b].at[j]], ssem[b], add=True))
            return hs

        @pl.loop(0, nwin // 2)
        def _w(wp):
            w0 = wp * 2
            hp0 = load_inputs(0, w0)
            hp1 = load_inputs(1, w0 + 1)
            compute_idx(0, hp0)
            hg0 = fire_gather(0)
            compute_idx(1, hp1)
            hg1 = fire_gather(1)
            hg0.wait()
            scale(0)
            hs0 = fire_scatters(0)
            hg1.wait()
            scale(1)
            hs1 = fire_scatters(1)
            for h in hs0 + hs1:
                h.wait()

        plsc.subcore_barrier()
        for j in range(nz):
            r0 = pl.multiple_of(s * ROWS_PER_SUB + j * zchunk, 8)
            pltpu.sync_copy(acc.at[pl.ds(r0, zchunk)],
                            out_hbm.at[c].at[pl.ds(r0, zchunk)])

    return body


_PASS_A = _make_sc_pass(2 * N, 16, jnp.float32, jnp.float32,
                        core_off=N, idx_mult=1, use_pat=False, win=512)
_PASS_B = _make_sc_pass(2 * N, 64, jnp.bfloat16, jnp.bfloat16,
                        core_off=N, idx_mult=1, use_pat=False, win=256)
_PASS_C = _make_sc_pass(2 * 7 * N, 32, jnp.bfloat16, jnp.bfloat16,
                        core_off=7 * N, idx_mult=7, use_pat=True, win=512)


def _dense_body(emb_ref, side_ref, w_ref, b_ref, wg_ref, bg_ref, out_ref):
    x = emb_ref[...] + side_ref[...]
    t = jnp.dot(x, w_ref[...], preferred_element_type=jnp.float32)
    t = t + b_ref[...][None, :]
    t = jnp.where(t >= 0, t, 0.01 * t)
    sc = jnp.dot(t, wg_ref[...], preferred_element_type=jnp.float32)
    out_ref[...] = sc + bg_ref[...][None, :]


def _loss_body(u_ref, p_ref, n_ref, eu_ref, ep_ref, en_ref, bpr_ref, reg_ref):
    u = u_ref[...]
    p = p_ref[...]
    n = n_ref[...]
    pos = jnp.sum(u * p, axis=1)
    neg = jnp.sum(u * n, axis=1)
    x = neg - pos
    sp = jnp.maximum(x, 0.0) + jnp.log1p(jnp.exp(-jnp.abs(x)))
    bpr_ref[...] = jnp.mean(sp).reshape(1, 1)
    reg = 0.5 * (jnp.sum(eu_ref[...] ** 2) + jnp.sum(ep_ref[...] ** 2)
                 + jnp.sum(en_ref[...] ** 2)) / B
    reg_ref[...] = (REG_LAMBDA * reg).reshape(1, 1)


def kernel(user, positive, negative, edge_index, edge_values, user_table,
           item_table, fc_W, fc_b, fcg_W, fcg_b):
    row = edge_index[0].astype(jnp.int32)
    col = edge_index[1].astype(jnp.int32)
    v = edge_values

    npad = E_PAD - E
    padi = jnp.arange(npad, dtype=jnp.int32) % N
    rowp = jnp.concatenate([row, padi])
    colp = jnp.concatenate([col, padi])
    vp = jnp.concatenate([v, jnp.zeros((npad,), jnp.float32)])
    row3d_256 = rowp.reshape(E_PAD // 256, 2, 128)
    row3d_512 = rowp.reshape(E_PAD // 512, 4, 128)

    emb = jnp.concatenate([user_table, item_table], axis=0)

    side_parts = []
    for q in range(2):
        tq = jnp.stack([emb[:, q * 32: q * 32 + 16],
                        emb[:, q * 32 + 16: q * 32 + 32]]).reshape(2 * N, 16)
        oq = _PASS_A(tq, colp, row3d_512, vp)
        side_parts += [oq[0, :N], oq[1, :N]]
    side = jnp.concatenate(side_parts, axis=1)

    wg8 = jnp.zeros((D, 8), jnp.float32).at[:, :G].set(fcg_W)
    bg8 = jnp.zeros((8,), jnp.float32).at[:G].set(fcg_b)
    scores8 = pl.pallas_call(
        _dense_body,
        grid=(25,),
        in_specs=[
            pl.BlockSpec((2000, D), lambda i: (i, 0)),
            pl.BlockSpec((2000, D), lambda i: (i, 0)),
            pl.BlockSpec((D, D), lambda i: (0, 0)),
            pl.BlockSpec((D,), lambda i: (0,)),
            pl.BlockSpec((D, 8), lambda i: (0, 0)),
            pl.BlockSpec((8,), lambda i: (0,)),
        ],
        out_specs=pl.BlockSpec((2000, 8), lambda i: (i, 0)),
        out_shape=jax.ShapeDtypeStruct((N, 8), jnp.float32),
    )(emb, side, fc_W, fc_b, wg8, bg8)

    scores = scores8[:, :G]
    top = jnp.max(scores, axis=1, keepdims=True)
    m = (scores == top).astype(jnp.float32)
    is_item = (jnp.arange(N) >= NUM_USERS)[:, None]
    m = jnp.where(is_item, 1.0, m)
    pat = (m[:, 0] + 2.0 * m[:, 1] + 4.0 * m[:, 2]).astype(jnp.int32) - 1

    mx = m[:, :, None] * emb[:, None, :]
    s_halves = []
    for q in range(2):
        tq = []
        for k in range(2):
            feats = mx[:, :, q * 32 + k * 16: q * 32 + (k + 1) * 16]
            rowt = jnp.concatenate(
                [feats[:, 0], feats[:, 1], feats[:, 2],
                 jnp.zeros((N, 16), jnp.float32)], axis=1)
            tq.append(rowt)
        table_q = jnp.stack(tq).reshape(2 * N, 64).astype(jnp.bfloat16)
        out_q = _PASS_B(table_q, colp, row3d_256, vp)
        s_halves.append(out_q[:, :N].astype(jnp.float32))

    s_parts = []
    for q in range(2):
        oq = s_halves[q][:, :, :48].reshape(2, N, 3, 16)
        s_parts.append(jnp.concatenate([oq[0], oq[1]], axis=2))
    s = jnp.concatenate(s_parts, axis=2)

    cur = m[:, :, None] * s
    layer1 = cur.sum(axis=1)

    c0, c1, c2 = cur[:, 0], cur[:, 1], cur[:, 2]
    curp = jnp.stack([c0, c1, c0 + c1, c2, c0 + c2, c1 + c2, c0 + c1 + c2],
                     axis=1)
    curp2 = jnp.stack([curp[:, :, :32], curp[:, :, 32:]])
    curp_tbl = curp2.reshape(2 * 7 * N, 32).astype(jnp.bfloat16)
    l2 = _PASS_C(curp_tbl, colp, row3d_512, vp, pat)
    layer2 = jnp.concatenate([l2[0, :N], l2[1, :N]],
                             axis=1).astype(jnp.float32)

    final = (G * emb + layer1 + layer2) * (1.0 / 3.0)
    users_emb, items_emb = final[:NUM_USERS], final[NUM_USERS:]

    u = users_emb[user]
    p = items_emb[positive]
    n = items_emb[negative]
    ego_u = user_table[user]
    ego_p = item_table[positive]
    ego_n = item_table[negative]

    bpr, reg = pl.pallas_call(
        _loss_body,
        out_shape=[jax.ShapeDtypeStruct((1, 1), jnp.float32),
                   jax.ShapeDtypeStruct((1, 1), jnp.float32)],
    )(u, p, n, ego_u, ego_p, ego_n)
    return (bpr[0, 0], reg[0, 0])
